# baseline (speedup 1.0000x reference)
import jax
import jax.numpy as jnp
from jax import lax
from jax.experimental import pallas as pl
from jax.experimental.pallas import tpu as pltpu

N = 10000
E = 320000
H = 128
ED = 16
EDP = 32

IR = 8
IC = 1600
EB = IR * IC
NSTEP = E // EB
NIR = E // IC

_R = 1000


def _scatter_body(src_ref, dst_ref, ea_ref, x_ref, s_out, a_out):
    @pl.when(pl.program_id(0) == 0)
    def _init():
        s_out[...] = jnp.zeros_like(s_out)
        a_out[...] = jnp.zeros_like(a_out)

    for a in range(IR):
        def body(j, _, a=a):
            s = src_ref[a, j]
            d = dst_ref[a, j]
            e = a * IC + j
            s_out[pl.ds(d, 1), :] = (s_out[pl.ds(d, 1), :]
                                     + x_ref[pl.ds(s, 1), :])
            a_out[pl.ds(d, 1), :] = (a_out[pl.ds(d, 1), :]
                                     + ea_ref[pl.ds(e, 1), :])
            return 0

        lax.fori_loop(0, IC, body, 0)


def _segment_sums(src, dst, ea_p, x):
    return pl.pallas_call(
        _scatter_body,
        grid=(NSTEP,),
        in_specs=[
            pl.BlockSpec((IR, IC), lambda i: (i, 0), memory_space=pltpu.SMEM),
            pl.BlockSpec((IR, IC), lambda i: (i, 0), memory_space=pltpu.SMEM),
            pl.BlockSpec((EB, EDP), lambda i: (i, 0)),
            pl.BlockSpec((N, H), lambda i: (0, 0)),
        ],
        out_specs=(
            pl.BlockSpec((N, H), lambda i: (0, 0)),
            pl.BlockSpec((N, EDP), lambda i: (0, 0)),
        ),
        out_shape=(
            jax.ShapeDtypeStruct((N, H), jnp.float32),
            jax.ShapeDtypeStruct((N, EDP), jnp.float32),
        ),
    )(src, dst, ea_p, x)


def _tc_body(x_ref, s_ref, a_ref, ws_ref, w1_ref, w2_ref, b_ref, o_ref):
    acc = jnp.dot(x_ref[...], ws_ref[...], preferred_element_type=jnp.float32)
    acc = acc + jnp.dot(s_ref[...], w1_ref[...],
                        preferred_element_type=jnp.float32)
    acc = acc + jnp.dot(a_ref[...], w2_ref[...],
                        preferred_element_type=jnp.float32)
    o_ref[...] = jnp.maximum(acc + b_ref[...], 0.0)


def _tc_dense(x, s_p, a_p, ws, w1, w2p, b2d):
    return pl.pallas_call(
        _tc_body,
        grid=(N // _R,),
        in_specs=[
            pl.BlockSpec((_R, H), lambda i: (i, 0)),
            pl.BlockSpec((_R, H), lambda i: (i, 0)),
            pl.BlockSpec((_R, EDP), lambda i: (i, 0)),
            pl.BlockSpec((H, H), lambda i: (0, 0)),
            pl.BlockSpec((H, H), lambda i: (0, 0)),
            pl.BlockSpec((EDP, H), lambda i: (0, 0)),
            pl.BlockSpec((1, H), lambda i: (0, 0)),
        ],
        out_specs=pl.BlockSpec((_R, H), lambda i: (i, 0)),
        out_shape=jax.ShapeDtypeStruct((N, H), jnp.float32),
    )(x, s_p, a_p, ws, w1, w2p, b2d)


def kernel(x, edge_index, edge_attr, W_self, b_self, W_msg, b_msg):
    src = edge_index[0].astype(jnp.int32).reshape(NIR, IC)
    dst = edge_index[1].astype(jnp.int32).reshape(NIR, IC)
    ea_p = jnp.concatenate(
        [edge_attr,
         jnp.ones((E, 1), jnp.float32),
         jnp.zeros((E, EDP - ED - 1), jnp.float32)], axis=1)
    s_p, a_p = _segment_sums(src, dst, ea_p, x)
    w1 = W_msg[:H]
    w2p = jnp.concatenate(
        [W_msg[H:], b_msg[None, :], jnp.zeros((EDP - ED - 1, H), jnp.float32)],
        axis=0)
    return _tc_dense(x, s_p, a_p, W_self, w1, w2p, b_self.reshape(1, H))
